# baseline (speedup 1.0000x reference)
import functools

import jax
import jax.numpy as jnp
from jax import lax
from jax.experimental import pallas as pl
from jax.experimental.pallas import tpu as pltpu
from jax.experimental.pallas import tpu_sc as plsc

B, L = 4096, 200
VOCAB, CAT = 100000, 1000
D = 128
H = 512
EPS = 1e-12
N = B * L

NCHUNK = 10
NT = N // NCHUNK

NC, NS = 2, 16
NW = NC * NS
PER_W = NT // NW
CH = 128
STEPS = PER_W // CH

BT = 4096


def _sc_gather_body(ids_hbm, cids_hbm, id_tab, cat_tab, out,
                    idx_a, cidx_a, idx_b, cidx_b, rows_a, rows_b,
                    s_ida, s_cata, s_idb, s_catb):
    wid = lax.axis_index("s") * NC + lax.axis_index("c")
    base_w = wid * PER_W

    def stage(i, idxbuf, cidxbuf):
        b = base_w + i * CH
        pltpu.sync_copy(ids_hbm.at[pl.ds(b, CH)], idxbuf)
        pltpu.sync_copy(cids_hbm.at[pl.ds(b, CH)], cidxbuf)

    def fire(idxbuf, cidxbuf, rows, sid, scat):
        pltpu.async_copy(id_tab.at[idxbuf], rows.at[:, pl.ds(0, D)], sid)
        pltpu.async_copy(cat_tab.at[cidxbuf], rows.at[:, pl.ds(D, D)], scat)

    def drain(idxbuf, cidxbuf, rows, sid, scat):
        pltpu.make_async_copy(id_tab.at[idxbuf], rows.at[:, pl.ds(0, D)],
                              sid).wait()
        pltpu.make_async_copy(cat_tab.at[cidxbuf], rows.at[:, pl.ds(D, D)],
                              scat).wait()

    def store(i, rows):
        b = base_w + i * CH
        pltpu.sync_copy(rows, out.at[pl.ds(b, CH)])

    stage(0, idx_a, cidx_a)
    fire(idx_a, cidx_a, rows_a, s_ida, s_cata)

    def body(j, carry):
        i0 = 2 * j
        stage(i0 + 1, idx_b, cidx_b)
        fire(idx_b, cidx_b, rows_b, s_idb, s_catb)
        drain(idx_a, cidx_a, rows_a, s_ida, s_cata)
        store(i0, rows_a)

        @pl.when(i0 + 2 < STEPS)
        def _refill():
            stage(i0 + 2, idx_a, cidx_a)
            fire(idx_a, cidx_a, rows_a, s_ida, s_cata)

        drain(idx_b, cidx_b, rows_b, s_idb, s_catb)
        store(i0 + 1, rows_b)
        return carry

    lax.fori_loop(0, STEPS // 2, body, 0)
    if STEPS % 2:
        drain(idx_a, cidx_a, rows_a, s_ida, s_cata)
        store(STEPS - 1, rows_a)


_sc_gather = functools.partial(
    pl.kernel,
    out_type=jax.ShapeDtypeStruct((NT, 2 * D), jnp.float32),
    mesh=plsc.VectorSubcoreMesh(core_axis_name="c", subcore_axis_name="s"),
    scratch_types=[
        pltpu.VMEM((CH,), jnp.int32),
        pltpu.VMEM((CH,), jnp.int32),
        pltpu.VMEM((CH,), jnp.int32),
        pltpu.VMEM((CH,), jnp.int32),
        pltpu.VMEM((CH, 2 * D), jnp.float32),
        pltpu.VMEM((CH, 2 * D), jnp.float32),
        pltpu.SemaphoreType.DMA,
        pltpu.SemaphoreType.DMA,
        pltpu.SemaphoreType.DMA,
        pltpu.SemaphoreType.DMA,
    ],
)(_sc_gather_body)


def _tc_body(y_ref, a_ref, w_ref, b_ref, g_ref, bt_ref, o_ref):
    del y_ref
    y = jnp.dot(a_ref[...], w_ref[...], preferred_element_type=jnp.float32)
    y = y + b_ref[...]
    mu = jnp.mean(y, axis=-1, keepdims=True)
    d = y - mu
    var = jnp.mean(d * d, axis=-1, keepdims=True)
    o_ref[...] = d * lax.rsqrt(var + EPS) * g_ref[...] + bt_ref[...]


def _tc_body0(a_ref, w_ref, b_ref, g_ref, bt_ref, o_ref):
    _tc_body(None, a_ref, w_ref, b_ref, g_ref, bt_ref, o_ref)


def _make_tc_call(k):
    base = k * (NT // BT)
    return pl.pallas_call(
        _tc_body if k else _tc_body0,
        grid=(NT // BT,),
        in_specs=([pl.BlockSpec(memory_space=pltpu.MemorySpace.HBM)]
                  if k else []) + [
            pl.BlockSpec((BT, 2 * D), lambda i: (i, 0)),
            pl.BlockSpec((2 * D, H), lambda i: (0, 0)),
            pl.BlockSpec((1, H), lambda i: (0, 0)),
            pl.BlockSpec((1, H), lambda i: (0, 0)),
            pl.BlockSpec((1, H), lambda i: (0, 0)),
        ],
        out_specs=pl.BlockSpec((BT, H), lambda i, base=base: (base + i, 0)),
        out_shape=jax.ShapeDtypeStruct((N, H), jnp.float32),
        input_output_aliases={0: 0} if k else {},
    )


_tc_calls = [_make_tc_call(k) for k in range(NCHUNK)]


def kernel(input_ids, category_ids, id_table, cat_table, W, b, gamma, beta):
    ids = input_ids.reshape(NCHUNK, NT)
    cids = category_ids.reshape(NCHUNK, NT)
    b2 = b.reshape(1, H)
    g2 = gamma.reshape(1, H)
    bt2 = beta.reshape(1, H)

    embs = [_sc_gather(ids[k], cids[k], id_table, cat_table)
            for k in range(NCHUNK)]
    y = _tc_calls[0](embs[0], W, b2, g2, bt2)
    for k in range(1, NCHUNK):
        y = _tc_calls[k](y, embs[k], W, b2, g2, bt2)
    return y.reshape(B, L, H)


# speedup vs baseline: 1.0215x; 1.0215x over previous
import functools

import jax
import jax.numpy as jnp
from jax import lax
from jax.experimental import pallas as pl
from jax.experimental.pallas import tpu as pltpu
from jax.experimental.pallas import tpu_sc as plsc

B, L = 4096, 200
VOCAB, CAT = 100000, 1000
D = 128
H = 512
EPS = 1e-12
N = B * L

NCHUNK = 8
NT = N // NCHUNK

NC, NS = 2, 16
NW = NC * NS
PER_W = NT // NW
CH = 128
STEPS = PER_W // CH

BT = 4096


def _sc_gather_body(ids_hbm, cids_hbm, id_tab, cat_tab, out,
                    idx_a, cidx_a, idx_b, cidx_b, rows_a, rows_b,
                    s_ida, s_cata, s_idb, s_catb):
    wid = lax.axis_index("s") * NC + lax.axis_index("c")
    base_w = wid * PER_W

    def stage(i, idxbuf, cidxbuf):
        b = base_w + i * CH
        pltpu.sync_copy(ids_hbm.at[pl.ds(b, CH)], idxbuf)
        pltpu.sync_copy(cids_hbm.at[pl.ds(b, CH)], cidxbuf)

    def fire(idxbuf, cidxbuf, rows, sid, scat):
        pltpu.async_copy(id_tab.at[idxbuf], rows.at[:, pl.ds(0, D)], sid)
        pltpu.async_copy(cat_tab.at[cidxbuf], rows.at[:, pl.ds(D, D)], scat)

    def drain(idxbuf, cidxbuf, rows, sid, scat):
        pltpu.make_async_copy(id_tab.at[idxbuf], rows.at[:, pl.ds(0, D)],
                              sid).wait()
        pltpu.make_async_copy(cat_tab.at[cidxbuf], rows.at[:, pl.ds(D, D)],
                              scat).wait()

    def store(i, rows):
        b = base_w + i * CH
        pltpu.sync_copy(rows, out.at[pl.ds(b, CH)])

    stage(0, idx_a, cidx_a)
    fire(idx_a, cidx_a, rows_a, s_ida, s_cata)

    def body(j, carry):
        i0 = 2 * j
        stage(i0 + 1, idx_b, cidx_b)
        fire(idx_b, cidx_b, rows_b, s_idb, s_catb)
        drain(idx_a, cidx_a, rows_a, s_ida, s_cata)
        store(i0, rows_a)

        @pl.when(i0 + 2 < STEPS)
        def _refill():
            stage(i0 + 2, idx_a, cidx_a)
            fire(idx_a, cidx_a, rows_a, s_ida, s_cata)

        drain(idx_b, cidx_b, rows_b, s_idb, s_catb)
        store(i0 + 1, rows_b)
        return carry

    lax.fori_loop(0, STEPS // 2, body, 0)
    if STEPS % 2:
        drain(idx_a, cidx_a, rows_a, s_ida, s_cata)
        store(STEPS - 1, rows_a)


_sc_gather = functools.partial(
    pl.kernel,
    out_type=jax.ShapeDtypeStruct((NT, 2 * D), jnp.float32),
    mesh=plsc.VectorSubcoreMesh(core_axis_name="c", subcore_axis_name="s"),
    scratch_types=[
        pltpu.VMEM((CH,), jnp.int32),
        pltpu.VMEM((CH,), jnp.int32),
        pltpu.VMEM((CH,), jnp.int32),
        pltpu.VMEM((CH,), jnp.int32),
        pltpu.VMEM((CH, 2 * D), jnp.float32),
        pltpu.VMEM((CH, 2 * D), jnp.float32),
        pltpu.SemaphoreType.DMA,
        pltpu.SemaphoreType.DMA,
        pltpu.SemaphoreType.DMA,
        pltpu.SemaphoreType.DMA,
    ],
)(_sc_gather_body)


def _tc_body(y_ref, a_ref, w_ref, b_ref, g_ref, bt_ref, o_ref):
    del y_ref
    y = jnp.dot(a_ref[...], w_ref[...], preferred_element_type=jnp.float32)
    y = y + b_ref[...]
    mu = jnp.mean(y, axis=-1, keepdims=True)
    d = y - mu
    var = jnp.mean(d * d, axis=-1, keepdims=True)
    o_ref[...] = d * lax.rsqrt(var + EPS) * g_ref[...] + bt_ref[...]


def _tc_body0(a_ref, w_ref, b_ref, g_ref, bt_ref, o_ref):
    _tc_body(None, a_ref, w_ref, b_ref, g_ref, bt_ref, o_ref)


def _make_tc_call(k):
    base = k * (NT // BT)
    return pl.pallas_call(
        _tc_body if k else _tc_body0,
        grid=(NT // BT,),
        in_specs=([pl.BlockSpec(memory_space=pltpu.MemorySpace.HBM)]
                  if k else []) + [
            pl.BlockSpec((BT, 2 * D), lambda i: (i, 0)),
            pl.BlockSpec((2 * D, H), lambda i: (0, 0)),
            pl.BlockSpec((1, H), lambda i: (0, 0)),
            pl.BlockSpec((1, H), lambda i: (0, 0)),
            pl.BlockSpec((1, H), lambda i: (0, 0)),
        ],
        out_specs=pl.BlockSpec((BT, H), lambda i, base=base: (base + i, 0)),
        out_shape=jax.ShapeDtypeStruct((N, H), jnp.float32),
        input_output_aliases={0: 0} if k else {},
    )


_tc_calls = [_make_tc_call(k) for k in range(NCHUNK)]


def kernel(input_ids, category_ids, id_table, cat_table, W, b, gamma, beta):
    ids = input_ids.reshape(NCHUNK, NT)
    cids = category_ids.reshape(NCHUNK, NT)
    b2 = b.reshape(1, H)
    g2 = gamma.reshape(1, H)
    bt2 = beta.reshape(1, H)

    embs = [_sc_gather(ids[k], cids[k], id_table, cat_table)
            for k in range(NCHUNK)]
    y = _tc_calls[0](embs[0], W, b2, g2, bt2)
    for k in range(1, NCHUNK):
        y = _tc_calls[k](y, embs[k], W, b2, g2, bt2)
    return y.reshape(B, L, H)
